# baseline (speedup 1.0000x reference)
import jax
import jax.numpy as jnp
from jax.experimental import pallas as pl

_KB = 2000


def _body(xT_ref, emb_ref, out_ref):
    p = jax.lax.dot_general(
        xT_ref[...].astype(jnp.bfloat16),
        emb_ref[...].astype(jnp.bfloat16),
        (((0,), (0,)), ((), ())),
        preferred_element_type=jnp.float32,
    )

    @pl.when(pl.program_id(0) == 0)
    def _():
        out_ref[...] = p

    @pl.when(pl.program_id(0) != 0)
    def _():
        out_ref[...] += p


def kernel(x_seq, emb):
    B, K = x_seq.shape
    H = emb.shape[1]
    return pl.pallas_call(
        _body,
        grid=(K // _KB,),
        in_specs=[
            pl.BlockSpec((_KB, B), lambda i: (i, 0)),
            pl.BlockSpec((_KB, H), lambda i: (i, 0)),
        ],
        out_specs=pl.BlockSpec((B, H), lambda i: (0, 0)),
        out_shape=jax.ShapeDtypeStruct((B, H), jnp.float32),
    )(x_seq.T, emb)
